# baseline (speedup 1.0000x reference)
import jax
import jax.numpy as jnp
from jax import lax
from jax.experimental import pallas as pl
from jax.experimental.pallas import tpu as pltpu

B, T, A_MAX, F = 32, 50, 94, 16
H = 128
A_PAD = 96
N = B * A_PAD
CW = 256
G_BLK = 8
NEG = float("-inf")


K_ROWS = F + H + 16
TSUB = 5


def _lstm_body(x_ref, w_ref, xh_ref, c_ref):
    t = pl.program_id(0)

    @pl.when(t == 0)
    def _init():
        xh_ref[F:F + H, :] = jnp.zeros((H, N), jnp.bfloat16)
        one_row = (lax.broadcasted_iota(jnp.int32, (16, N), 0) == 0)
        xh_ref[F + H:, :] = one_row.astype(jnp.bfloat16)
        c_ref[...] = jnp.zeros((H, N), jnp.float32)

    w = w_ref[...]

    for s in range(TSUB):
        xh_ref[0:F, :] = x_ref[s].astype(jnp.bfloat16)
        for j in range(N // CW):
            cols = pl.ds(j * CW, CW)
            xh = xh_ref[:, cols]
            ti = jnp.tanh(lax.dot_general(w[0 * H:1 * H], xh, (((1,), (0,)), ((), ())),
                                          preferred_element_type=jnp.float32))
            tf = jnp.tanh(lax.dot_general(w[1 * H:2 * H], xh, (((1,), (0,)), ((), ())),
                                          preferred_element_type=jnp.float32))
            tg = jnp.tanh(lax.dot_general(w[2 * H:3 * H], xh, (((1,), (0,)), ((), ())),
                                          preferred_element_type=jnp.float32))
            to = jnp.tanh(lax.dot_general(w[3 * H:4 * H], xh, (((1,), (0,)), ((), ())),
                                          preferred_element_type=jnp.float32))
            c_old = c_ref[:, cols]
            c = 0.5 * ((c_old + tg) + (tf * c_old + ti * tg))
            c_ref[:, cols] = c
            tc = jnp.tanh(c)
            h2 = tc + to * tc
            xh_ref[F:F + H, cols] = h2.astype(jnp.bfloat16)


def _sage_layer(hn, wmask3, wp, bp, ws, wn, b):
    m = jnp.maximum(hn @ wp + bp, 0.0)
    m3 = m.reshape(G_BLK, A_PAD, H)
    mv = jnp.where(wmask3 > 0, m3, NEG)
    t1 = jnp.max(mv, axis=1, keepdims=True)
    ismax = mv == t1
    iota = lax.broadcasted_iota(jnp.int32, (G_BLK, A_PAD, H), 1)
    k = jnp.where(ismax, iota, A_PAD + 1)
    kmin = jnp.min(k, axis=1, keepdims=True)
    first = iota == kmin
    t2 = jnp.max(jnp.where(first, NEG, mv), axis=1, keepdims=True)
    agg = jnp.where(ismax, t2, t1)
    agg2 = agg.reshape(G_BLK * A_PAD, H)
    return hn @ ws + agg2 @ wn + b


def _gnn_body(xh_ref, nper_ref, ht_ref,
              wp1_ref, bp1_ref, ws1_ref, wn1_ref, b1_ref,
              wp2_ref, bp2_ref, ws2_ref, wn2_ref, b2_ref, out_ref):
    half_ident = (0.5 * (lax.broadcasted_iota(jnp.int32, (H, H), 0)
                         == lax.broadcasted_iota(jnp.int32, (H, H), 1)
                         ).astype(jnp.float32)).astype(jnp.bfloat16)
    hn = lax.dot_general(xh_ref[F:F + H, :], half_ident,
                         (((0,), (0,)), ((), ())),
                         preferred_element_type=jnp.float32)
    nper3 = nper_ref[...].reshape(G_BLK, 1, 1)
    fi = lax.broadcasted_iota(jnp.int32, (G_BLK, A_PAD, H), 1).astype(jnp.float32)
    wmask3 = jnp.where(fi < nper3, 1.0 / nper3, 0.0)
    h1 = jnp.tanh(_sage_layer(hn, wmask3, wp1_ref[...], bp1_ref[...],
                              ws1_ref[...], wn1_ref[...], b1_ref[...]))
    h2 = _sage_layer(h1, wmask3, wp2_ref[...], bp2_ref[...],
                     ws2_ref[...], wn2_ref[...], b2_ref[...])
    h2w = h2 * wmask3.reshape(G_BLK * A_PAD, H)
    out_ref[:, 0:H] = jnp.sum(h2w.reshape(G_BLK, A_PAD, H), axis=1)
    out_ref[:, H:H + 3] = ht_ref[...]


def kernel(agent_obs, hideout_obs, timestep_obs, num_agents, W_ih, W_hh,
           b_ih, b_hh, W_pool1, b_pool1, W_self1, W_neigh1, b1,
           W_pool2, b_pool2, W_self2, W_neigh2, b2):
    n_per = num_agents + 83
    x = jnp.pad(agent_obs, ((0, 0), (0, 0), (0, A_PAD - A_MAX), (0, 0)))
    xT = jnp.transpose(x, (1, 3, 0, 2)).reshape(T, F, N)
    rs = jnp.concatenate([jnp.full((H,), 0.5), jnp.full((H,), 0.5),
                          jnp.ones((H,)), jnp.full((H,), 0.5)]).astype(jnp.float32)
    w_cat = jnp.concatenate([
        rs[:, None] * W_ih,
        (0.5 * rs)[:, None] * W_hh,
        (rs * (b_ih + b_hh))[:, None],
        jnp.zeros((4 * H, 15), jnp.float32),
    ], axis=1).astype(jnp.bfloat16)

    xh = pl.pallas_call(
        _lstm_body,
        grid=(T // TSUB,),
        in_specs=[
            pl.BlockSpec((TSUB, F, N), lambda t: (t, 0, 0)),
            pl.BlockSpec((4 * H, K_ROWS), lambda t: (0, 0)),
        ],
        out_specs=pl.BlockSpec((K_ROWS, N), lambda t: (0, 0)),
        out_shape=jax.ShapeDtypeStruct((K_ROWS, N), jnp.bfloat16),
        scratch_shapes=[
            pltpu.VMEM((H, N), jnp.float32),
        ],
        compiler_params=pltpu.CompilerParams(
            dimension_semantics=("arbitrary",)),
    )(xT, w_cat)

    nperf = n_per.astype(jnp.float32)[:, None]
    ht = jnp.concatenate([hideout_obs, timestep_obs], axis=1)

    R = G_BLK * A_PAD
    out = pl.pallas_call(
        _gnn_body,
        grid=(B // G_BLK,),
        in_specs=[
            pl.BlockSpec((K_ROWS, R), lambda j: (0, j)),
            pl.BlockSpec((G_BLK, 1), lambda j: (j, 0)),
            pl.BlockSpec((G_BLK, 3), lambda j: (j, 0)),
            pl.BlockSpec((H, H), lambda j: (0, 0)),
            pl.BlockSpec((1, H), lambda j: (0, 0)),
            pl.BlockSpec((H, H), lambda j: (0, 0)),
            pl.BlockSpec((H, H), lambda j: (0, 0)),
            pl.BlockSpec((1, H), lambda j: (0, 0)),
            pl.BlockSpec((H, H), lambda j: (0, 0)),
            pl.BlockSpec((1, H), lambda j: (0, 0)),
            pl.BlockSpec((H, H), lambda j: (0, 0)),
            pl.BlockSpec((H, H), lambda j: (0, 0)),
            pl.BlockSpec((1, H), lambda j: (0, 0)),
        ],
        out_specs=pl.BlockSpec((G_BLK, H + 3), lambda j: (j, 0)),
        out_shape=jax.ShapeDtypeStruct((B, H + 3), jnp.float32),
        compiler_params=pltpu.CompilerParams(
            dimension_semantics=("arbitrary",)),
    )(xh, nperf, ht,
      W_pool1.T, b_pool1[None, :], W_self1.T, W_neigh1.T, b1[None, :],
      W_pool2.T, b_pool2[None, :], W_self2.T, W_neigh2.T, b2[None, :])

    return out


# speedup vs baseline: 1.2803x; 1.2803x over previous
import jax
import jax.numpy as jnp
from jax import lax
from jax.experimental import pallas as pl
from jax.experimental.pallas import tpu as pltpu

B, T, A_MAX, F = 32, 50, 94, 16
H = 128
A_PAD = 96
N = B * A_PAD
CW = 384
G_BLK = 8
NEG = float("-inf")


K_ROWS = F + H + 16
TSUB = 5


def _lstm_body(x_ref, w_ref, xh_ref, c_ref):
    t = pl.program_id(0)

    @pl.when(t == 0)
    def _init():
        xh_ref[F:F + H, :] = jnp.zeros((H, N), jnp.bfloat16)
        one_row = (lax.broadcasted_iota(jnp.int32, (16, N), 0) == 0)
        xh_ref[F + H:, :] = one_row.astype(jnp.bfloat16)
        c_ref[...] = jnp.zeros((H, N), jnp.float32)

    w = w_ref[...]

    for s in range(TSUB):
        xh_ref[0:F, :] = x_ref[s]
        for j in range(N // CW):
            cols = pl.ds(j * CW, CW)
            xh = xh_ref[:, cols]
            ti = jnp.tanh(lax.dot_general(w[0 * H:1 * H], xh, (((1,), (0,)), ((), ())),
                                          preferred_element_type=jnp.float32))
            tf = jnp.tanh(lax.dot_general(w[1 * H:2 * H], xh, (((1,), (0,)), ((), ())),
                                          preferred_element_type=jnp.float32))
            tg = jnp.tanh(lax.dot_general(w[2 * H:3 * H], xh, (((1,), (0,)), ((), ())),
                                          preferred_element_type=jnp.float32))
            to = jnp.tanh(lax.dot_general(w[3 * H:4 * H], xh, (((1,), (0,)), ((), ())),
                                          preferred_element_type=jnp.float32))
            c_old = c_ref[:, cols]
            c = 0.5 * ((c_old + tg) + (tf * c_old + ti * tg))
            c_ref[:, cols] = c
            tc = jnp.tanh(c)
            h2 = tc + to * tc
            xh_ref[F:F + H, cols] = h2.astype(jnp.bfloat16)


def _sage_layer(hn, wmask3, wp, bp, ws, wn, b):
    m = jnp.maximum(hn @ wp + bp, 0.0)
    m3 = m.reshape(G_BLK, A_PAD, H)
    mv = jnp.where(wmask3 > 0, m3, NEG)
    t1 = jnp.max(mv, axis=1, keepdims=True)
    ismax = mv == t1
    iota = lax.broadcasted_iota(jnp.int32, (G_BLK, A_PAD, H), 1)
    k = jnp.where(ismax, iota, A_PAD + 1)
    kmin = jnp.min(k, axis=1, keepdims=True)
    first = iota == kmin
    t2 = jnp.max(jnp.where(first, NEG, mv), axis=1, keepdims=True)
    agg = jnp.where(ismax, t2, t1)
    agg2 = agg.reshape(G_BLK * A_PAD, H)
    return hn @ ws + agg2 @ wn + b


def _gnn_body(xh_ref, nper_ref, ht_ref,
              wp1_ref, bp1_ref, ws1_ref, wn1_ref, b1_ref,
              wp2_ref, bp2_ref, ws2_ref, wn2_ref, b2_ref, out_ref):
    half_ident = (0.5 * (lax.broadcasted_iota(jnp.int32, (H, H), 0)
                         == lax.broadcasted_iota(jnp.int32, (H, H), 1)
                         ).astype(jnp.float32)).astype(jnp.bfloat16)
    hn = lax.dot_general(xh_ref[F:F + H, :], half_ident,
                         (((0,), (0,)), ((), ())),
                         preferred_element_type=jnp.float32)
    nper3 = nper_ref[...].reshape(G_BLK, 1, 1)
    fi = lax.broadcasted_iota(jnp.int32, (G_BLK, A_PAD, H), 1).astype(jnp.float32)
    wmask3 = jnp.where(fi < nper3, 1.0 / nper3, 0.0)
    h1 = jnp.tanh(_sage_layer(hn, wmask3, wp1_ref[...], bp1_ref[...],
                              ws1_ref[...], wn1_ref[...], b1_ref[...]))
    h2 = _sage_layer(h1, wmask3, wp2_ref[...], bp2_ref[...],
                     ws2_ref[...], wn2_ref[...], b2_ref[...])
    h2w = h2 * wmask3.reshape(G_BLK * A_PAD, H)
    out_ref[:, 0:H] = jnp.sum(h2w.reshape(G_BLK, A_PAD, H), axis=1)
    out_ref[:, H:H + 3] = ht_ref[...]


def kernel(agent_obs, hideout_obs, timestep_obs, num_agents, W_ih, W_hh,
           b_ih, b_hh, W_pool1, b_pool1, W_self1, W_neigh1, b1,
           W_pool2, b_pool2, W_self2, W_neigh2, b2):
    n_per = num_agents + 83
    x = jnp.pad(agent_obs.astype(jnp.bfloat16),
                ((0, 0), (0, 0), (0, A_PAD - A_MAX), (0, 0)))
    xT = jnp.transpose(x, (1, 3, 0, 2)).reshape(T, F, N)
    rs = jnp.concatenate([jnp.full((H,), 0.5), jnp.full((H,), 0.5),
                          jnp.ones((H,)), jnp.full((H,), 0.5)]).astype(jnp.float32)
    w_cat = jnp.concatenate([
        rs[:, None] * W_ih,
        (0.5 * rs)[:, None] * W_hh,
        (rs * (b_ih + b_hh))[:, None],
        jnp.zeros((4 * H, 15), jnp.float32),
    ], axis=1).astype(jnp.bfloat16)

    xh = pl.pallas_call(
        _lstm_body,
        grid=(T // TSUB,),
        in_specs=[
            pl.BlockSpec((TSUB, F, N), lambda t: (t, 0, 0)),
            pl.BlockSpec((4 * H, K_ROWS), lambda t: (0, 0)),
        ],
        out_specs=pl.BlockSpec((K_ROWS, N), lambda t: (0, 0)),
        out_shape=jax.ShapeDtypeStruct((K_ROWS, N), jnp.bfloat16),
        scratch_shapes=[
            pltpu.VMEM((H, N), jnp.float32),
        ],
        compiler_params=pltpu.CompilerParams(
            dimension_semantics=("arbitrary",)),
    )(xT, w_cat)

    nperf = n_per.astype(jnp.float32)[:, None]
    ht = jnp.concatenate([hideout_obs, timestep_obs], axis=1)

    R = G_BLK * A_PAD
    out = pl.pallas_call(
        _gnn_body,
        grid=(B // G_BLK,),
        in_specs=[
            pl.BlockSpec((K_ROWS, R), lambda j: (0, j)),
            pl.BlockSpec((G_BLK, 1), lambda j: (j, 0)),
            pl.BlockSpec((G_BLK, 3), lambda j: (j, 0)),
            pl.BlockSpec((H, H), lambda j: (0, 0)),
            pl.BlockSpec((1, H), lambda j: (0, 0)),
            pl.BlockSpec((H, H), lambda j: (0, 0)),
            pl.BlockSpec((H, H), lambda j: (0, 0)),
            pl.BlockSpec((1, H), lambda j: (0, 0)),
            pl.BlockSpec((H, H), lambda j: (0, 0)),
            pl.BlockSpec((1, H), lambda j: (0, 0)),
            pl.BlockSpec((H, H), lambda j: (0, 0)),
            pl.BlockSpec((H, H), lambda j: (0, 0)),
            pl.BlockSpec((1, H), lambda j: (0, 0)),
        ],
        out_specs=pl.BlockSpec((G_BLK, H + 3), lambda j: (j, 0)),
        out_shape=jax.ShapeDtypeStruct((B, H + 3), jnp.float32),
        compiler_params=pltpu.CompilerParams(
            dimension_semantics=("arbitrary",)),
    )(xh, nperf, ht,
      W_pool1.T, b_pool1[None, :], W_self1.T, W_neigh1.T, b1[None, :],
      W_pool2.T, b_pool2[None, :], W_self2.T, W_neigh2.T, b2[None, :])

    return out
